# baseline (speedup 1.0000x reference)
import jax
import jax.numpy as jnp
from jax import lax
from jax.experimental import pallas as pl
from jax.experimental.pallas import tpu as pltpu
from jax.experimental.pallas import tpu_sc as plsc

B = 16384
D = 64
PACK = 2 * D
NC = 2
NS = 16
NW = NC * NS
BPW = B // NW
CHUNK = 128
NCHUNK = BPW // CHUNK
L = 16


def _rsqrt(x):
    i = plsc.bitcast(x, jnp.int32)
    i = jnp.int32(0x5F3759DF) - lax.shift_right_arithmetic(i, 1)
    y = plsc.bitcast(i, jnp.float32)
    for _ in range(3):
        y = y * (1.5 - 0.5 * x * y * y)
    return y


def _sc_kernel(heads_hbm, rels_hbm, tails_hbm, node_hbm, rel_hbm, out_hbm,
               hidx, ridx, tidx, ph, pr, pt, oh, orr, ot,
               hbuf, rbuf, tbuf, obuf, sem):
    wid = lax.axis_index("s") * NC + lax.axis_index("c")
    base = wid * BPW

    pltpu.sync_copy(heads_hbm.at[wid], hidx)
    pltpu.sync_copy(rels_hbm.at[wid], ridx)
    pltpu.sync_copy(tails_hbm.at[wid], tidx)

    for idx_ref, p_ref, o_ref in ((hidx, ph, oh), (ridx, pr, orr), (tidx, pt, ot)):
        for j in range(NCHUNK):
            for k in range(CHUNK // L):
                v = idx_ref[j, pl.ds(k * L, L)]
                p_ref[j, pl.ds(k * L, L)] = lax.shift_right_logical(v, 1)
                o_ref[j, pl.ds(k * L, L)] = lax.shift_left(v & 1, 6)

    lanes = lax.iota(jnp.int32, L)
    zero = jnp.zeros((L,), jnp.float32)

    for j in range(NCHUNK):
        ch = pltpu.async_copy(node_hbm.at[ph.at[j]], hbuf, sem)
        cr = pltpu.async_copy(rel_hbm.at[pr.at[j]], rbuf, sem)
        ct = pltpu.async_copy(node_hbm.at[pt.at[j]], tbuf, sem)
        ch.wait(); cr.wait(); ct.wait()

        jv = jnp.full((L,), j, jnp.int32)

        def group(g, _):
            rows = g * L + lanes
            ohv = plsc.load_gather(oh, [jv, rows])
            orv = plsc.load_gather(orr, [jv, rows])
            otv = plsc.load_gather(ot, [jv, rows])
            nh = zero; nt = zero; nr = zero
            hr = zero; ht = zero; rt = zero
            for d in range(D):
                gh = plsc.load_gather(hbuf, [rows, ohv + d])
                gr = plsc.load_gather(rbuf, [rows, orv + d])
                gt = plsc.load_gather(tbuf, [rows, otv + d])
                nh = nh + gh * gh
                nt = nt + gt * gt
                nr = nr + gr * gr
                hr = hr + gh * gr
                ht = ht + gh * gt
                rt = rt + gr * gt
            ih = _rsqrt(jnp.maximum(nh, 1e-24))
            it = _rsqrt(jnp.maximum(nt, 1e-24))
            s = (nh * ih * ih + nr + nt * it * it
                 + 2.0 * hr * ih - 2.0 * (ht * ih) * it - 2.0 * rt * it)
            s = jnp.maximum(s, 0.0)
            score = -(s * _rsqrt(jnp.maximum(s, 1e-30)))
            plsc.store_scatter(obuf, [j * CHUNK + rows], score)
            return _

        lax.fori_loop(0, CHUNK // L, group, None)

    pltpu.sync_copy(obuf, out_hbm.at[pl.ds(base, BPW)])


@jax.jit
def kernel(batched_paths, node_emb, rel_emb):
    heads = batched_paths[:, 2].reshape(NW, NCHUNK, CHUNK)
    rels = batched_paths[:, 1].reshape(NW, NCHUNK, CHUNK)
    tails = batched_paths[:, 0].reshape(NW, NCHUNK, CHUNK)
    node_p = node_emb.reshape(-1, PACK)
    rel_p = rel_emb.reshape(-1, PACK)

    mesh = plsc.VectorSubcoreMesh(core_axis_name="c", subcore_axis_name="s",
                                  num_cores=NC, num_subcores=NS)
    run = pl.kernel(
        _sc_kernel,
        out_type=jax.ShapeDtypeStruct((B,), jnp.float32),
        mesh=mesh,
        compiler_params=pltpu.CompilerParams(needs_layout_passes=False),
        scratch_types=[
            pltpu.VMEM((NCHUNK, CHUNK), jnp.int32),
            pltpu.VMEM((NCHUNK, CHUNK), jnp.int32),
            pltpu.VMEM((NCHUNK, CHUNK), jnp.int32),
            pltpu.VMEM((NCHUNK, CHUNK), jnp.int32),
            pltpu.VMEM((NCHUNK, CHUNK), jnp.int32),
            pltpu.VMEM((NCHUNK, CHUNK), jnp.int32),
            pltpu.VMEM((NCHUNK, CHUNK), jnp.int32),
            pltpu.VMEM((NCHUNK, CHUNK), jnp.int32),
            pltpu.VMEM((NCHUNK, CHUNK), jnp.int32),
            pltpu.VMEM((CHUNK, PACK), jnp.float32),
            pltpu.VMEM((CHUNK, PACK), jnp.float32),
            pltpu.VMEM((CHUNK, PACK), jnp.float32),
            pltpu.VMEM((BPW,), jnp.float32),
            pltpu.SemaphoreType.DMA,
        ],
    )
    return run(heads, rels, tails, node_p, rel_p)


# speedup vs baseline: 2.6853x; 2.6853x over previous
import jax
import jax.numpy as jnp
from jax import lax
from jax.experimental import pallas as pl
from jax.experimental.pallas import tpu as pltpu
from jax.experimental.pallas import tpu_sc as plsc

B = 16384
D = 64
V = 1_000_000
NC = 2
NS = 16
NDH = D // NC
BPT = B // NS
L = 16
NQ = 6


def _rsqrt(x):
    i = plsc.bitcast(x, jnp.int32)
    i = jnp.int32(0x5F3759DF) - lax.shift_right_arithmetic(i, 1)
    y = plsc.bitcast(i, jnp.float32)
    for _ in range(3):
        y = y * (1.5 - 0.5 * x * y * y)
    return y


TCUT = 999936
CH = 8
NCHK = NDH // CH


def _partials_kernel(heads_hbm, rels_hbm, tails_hbm, node_hbm, rel_hbm,
                     ntail_hbm, rtail_hbm, part_hbm, *scratch):
    (hidx, ridx, tidx, hg, rg, tg, hof, rof, tof,
     nhb, ntb, nrb, hrb, htb, rtb, tailn, tailr) = scratch[:17]
    hstg = scratch[17:17 + CH]
    tstg = scratch[17 + CH:17 + 2 * CH]
    rstg = scratch[17 + 2 * CH:17 + 3 * CH]
    slab, gsem, ssem = scratch[17 + 3 * CH:]

    c = lax.axis_index("c")
    s = lax.axis_index("s")
    dbase = c * NDH

    pltpu.sync_copy(heads_hbm.at[s], hidx)
    pltpu.sync_copy(rels_hbm.at[s], ridx)
    pltpu.sync_copy(tails_hbm.at[s], tidx)
    pltpu.sync_copy(ntail_hbm, tailn)
    pltpu.sync_copy(rtail_hbm, tailr)

    def prep(v, _):
        sl = pl.ds(v * L, L)
        for iref, gref, oref in ((hidx, hg, hof), (ridx, rg, rof),
                                 (tidx, tg, tof)):
            iv = iref[sl]
            gref[sl] = jnp.minimum(iv, TCUT - 1)
            oref[sl] = jnp.maximum(iv - TCUT, 0)
        zeros = jnp.zeros((L,), jnp.float32)
        for aref in (nhb, ntb, nrb, hrb, htb, rtb):
            aref[sl] = zeros
        return _

    lax.fori_loop(0, BPT // L, prep, None)

    SEG = 62464
    REM = TCUT - 16 * SEG

    def stream_dim(tab_hbm, dl, gather_specs, pre_compute=None):
        dd = dbase + dl
        off = s * SEG
        cp_seg = pltpu.async_copy(tab_hbm.at[dd, pl.ds(off, SEG)],
                                  slab.at[pl.ds(off, SEG)], ssem)

        @pl.when(s == 0)
        def _():
            pltpu.sync_copy(tab_hbm.at[dd, pl.ds(16 * SEG, REM)],
                            slab.at[pl.ds(16 * SEG, REM)])

        if pre_compute is not None:
            pre_compute()
        cp_seg.wait()
        plsc.subcore_barrier()
        copies = [pltpu.async_copy(slab.at[idx_ref], buf, gsem)
                  for idx_ref, buf in gather_specs]
        for cp in copies:
            cp.wait()
        plsc.subcore_barrier()

    lanes = lax.iota(jnp.int32, L)
    zero = jnp.zeros((L,), jnp.float32)

    def make_acc(chunk):
        dvs = [jnp.full((L,), dbase + chunk * CH + j, jnp.int32)
               for j in range(CH)]

        def acc_vec(v, _):
            sl = pl.ds(v * L, L)
            mh = hidx[sl] >= TCUT
            mt = tidx[sl] >= TCUT
            mr = ridx[sl] >= TCUT
            hov = hof[sl]
            tov = tof[sl]
            rov = rof[sl]
            nh = zero; nt = zero; nr = zero
            hr = zero; ht = zero; rt = zero
            for j in range(CH):
                gh = jnp.where(mh, plsc.load_gather(tailn, [dvs[j], hov]),
                               hstg[j][sl])
                gt = jnp.where(mt, plsc.load_gather(tailn, [dvs[j], tov]),
                               tstg[j][sl])
                gr = jnp.where(mr, plsc.load_gather(tailr, [dvs[j], rov]),
                               rstg[j][sl])
                nh = nh + gh * gh
                nt = nt + gt * gt
                nr = nr + gr * gr
                hr = hr + gh * gr
                ht = ht + gh * gt
                rt = rt + gr * gt
            for aref, val in ((nhb, nh), (ntb, nt), (nrb, nr),
                              (hrb, hr), (htb, ht), (rtb, rt)):
                aref[sl] = aref[sl] + val
            return _

        return lambda: lax.fori_loop(0, BPT // L, acc_vec, None)

    pending_acc = None
    for chunk in range(NCHK):
        for j in range(CH):
            stream_dim(node_hbm, chunk * CH + j,
                       [(hg, hstg[j]), (tg, tstg[j])],
                       pre_compute=(pending_acc if j == 0 else None))
            pending_acc = None
        for j in range(CH):
            stream_dim(rel_hbm, chunk * CH + j, [(rg, rstg[j])])
        pending_acc = make_acc(chunk)
    pending_acc()

    for q, ref in enumerate((nhb, ntb, nrb, hrb, htb, rtb)):
        pltpu.sync_copy(ref, part_hbm.at[c, q, pl.ds(s * BPT, BPT)])


def _combine_kernel(part_hbm, out_hbm, pbuf, obuf):
    wid = lax.axis_index("s") * NC + lax.axis_index("c")
    bpw = B // (NC * NS)
    base = wid * bpw
    for cc in range(NC):
        for q in range(NQ):
            pltpu.sync_copy(part_hbm.at[cc, q, pl.ds(base, bpw)],
                            pbuf.at[cc * NQ + q])

    lanes = lax.iota(jnp.int32, L)

    def vec(v, _):
        sl = pl.ds(v * L, L)
        nh = pbuf[0, sl] + pbuf[NQ, sl]
        nt = pbuf[1, sl] + pbuf[NQ + 1, sl]
        nr = pbuf[2, sl] + pbuf[NQ + 2, sl]
        hr = pbuf[3, sl] + pbuf[NQ + 3, sl]
        ht = pbuf[4, sl] + pbuf[NQ + 4, sl]
        rt = pbuf[5, sl] + pbuf[NQ + 5, sl]
        ih = _rsqrt(jnp.maximum(nh, 1e-24))
        it = _rsqrt(jnp.maximum(nt, 1e-24))
        ss = (nh * ih * ih + nr + nt * it * it
              + 2.0 * hr * ih - 2.0 * (ht * ih) * it - 2.0 * rt * it)
        ss = jnp.maximum(ss, 0.0)
        score = -(ss * _rsqrt(jnp.maximum(ss, 1e-30)))
        plsc.store_scatter(obuf, [v * L + lanes], score)
        return _

    lax.fori_loop(0, bpw // L, vec, None)
    pltpu.sync_copy(obuf, out_hbm.at[pl.ds(base, bpw)])


@jax.jit
def kernel(batched_paths, node_emb, rel_emb):
    heads = batched_paths[:, 2].reshape(NS, BPT)
    rels = batched_paths[:, 1].reshape(NS, BPT)
    tails = batched_paths[:, 0].reshape(NS, BPT)
    node_t = node_emb.T
    rel_t = rel_emb.T
    ntail = node_emb[999936:, :].T
    rtail = rel_emb[999936:, :].T

    mesh = plsc.VectorSubcoreMesh(core_axis_name="c", subcore_axis_name="s",
                                  num_cores=NC, num_subcores=NS)
    parts = pl.kernel(
        _partials_kernel,
        out_type=jax.ShapeDtypeStruct((NC, NQ, B), jnp.float32),
        mesh=mesh,
        compiler_params=pltpu.CompilerParams(needs_layout_passes=False),
        scratch_types=(
            [pltpu.VMEM((BPT,), jnp.int32)] * 9
            + [pltpu.VMEM((BPT,), jnp.float32)] * NQ
            + [pltpu.VMEM((D, 64), jnp.float32)] * 2
            + [pltpu.VMEM((BPT,), jnp.float32)] * (3 * CH)
            + [pltpu.VMEM_SHARED((TCUT,), jnp.float32)]
            + [pltpu.SemaphoreType.DMA, pltpu.SemaphoreType.DMA]
        ),
    )(heads, rels, tails, node_t, rel_t, ntail, rtail)

    out = pl.kernel(
        _combine_kernel,
        out_type=jax.ShapeDtypeStruct((B,), jnp.float32),
        mesh=mesh,
        compiler_params=pltpu.CompilerParams(needs_layout_passes=False),
        scratch_types=[
            pltpu.VMEM((NC * NQ, B // (NC * NS)), jnp.float32),
            pltpu.VMEM((B // (NC * NS),), jnp.float32),
        ],
    )(parts)
    return out
